# baseline (speedup 1.0000x reference)
import functools

import jax
import jax.numpy as jnp
from jax import lax
from jax.experimental import pallas as pl
from jax.experimental.pallas import tpu as pltpu
from jax.experimental.pallas import tpu_sc as plsc

NUM_CONSTANTS = 1000000
C_DIM = 32
N_X = 100000
T_PER_PRED = 65536
T = 2 * T_PER_PRED

NC, NS = 2, 16
NW = NC * NS

BLK = 128
N_BLOCKS = T // BLK
BLOCKS_PER_W = N_BLOCKS // NW
X_PAD = 100352

_MESH = plsc.VectorSubcoreMesh(
    core_axis_name="c", subcore_axis_name="s", num_cores=NC, num_subcores=NS)
_SC_PARAMS = pltpu.CompilerParams(
    use_tc_tiling_on_sc=False, needs_layout_passes=False)


_S1_COLS = 2048
_S1_NCOL = NUM_CONSTANTS // _S1_COLS
_S1_GRID = 492
_S1_OUTROWS = (_S1_GRID // 4) * _S1_COLS
PROJ_ROWS = 4 * _S1_OUTROWS
_TAIL = NUM_CONSTANTS - _S1_NCOL * _S1_COLS


def _project_body(xt_ref, tail_ref, w_ref, p_ref, wp_ref, bp_ref, o_ref, r2_ref):
    g = pl.program_id(0)
    x = jnp.where(g >= _S1_NCOL, tail_ref[...], xt_ref[...])
    part = jax.lax.dot_general(
        x, w_ref[...], (((0,), (0,)), ((), ())),
        preferred_element_type=jnp.float32)
    for a in range(4):
        @pl.when(g % 4 == a)
        def _():
            o_ref[:, 32 * a:32 * a + 32] = part

    @pl.when(g == 0)
    def _():
        r2_ref[...] = (
            jnp.dot(p_ref[...], wp_ref[...], preferred_element_type=jnp.float32)
            + bp_ref[...]
        )


_project = pl.pallas_call(
    _project_body,
    grid=(_S1_GRID,),
    in_specs=[
        pl.BlockSpec((C_DIM, _S1_COLS),
                     lambda g: (0, jnp.minimum(g, _S1_NCOL - 1))),
        pl.BlockSpec((C_DIM, _S1_COLS), lambda g: (0, 0)),
        pl.BlockSpec((C_DIM, C_DIM), lambda g: (0, 0)),
        pl.BlockSpec((2, C_DIM), lambda g: (0, 0)),
        pl.BlockSpec((C_DIM, C_DIM), lambda g: (0, 0)),
        pl.BlockSpec((1, C_DIM), lambda g: (0, 0)),
    ],
    out_specs=[
        pl.BlockSpec((_S1_COLS, 128), lambda g: (g // 4, 0)),
        pl.BlockSpec((2, C_DIM), lambda g: (0, 0)),
    ],
    out_shape=[
        jax.ShapeDtypeStruct((_S1_OUTROWS, 128), jnp.float32),
        jax.ShapeDtypeStruct((2, C_DIM), jnp.float32),
    ],
)


@functools.partial(
    pl.kernel,
    out_type=jax.ShapeDtypeStruct((T, C_DIM), jnp.float32),
    mesh=_MESH,
    compiler_params=_SC_PARAMS,
    scratch_types=[
        pltpu.VMEM((X_PAD,), jnp.int32),
        pltpu.VMEM((2, BLK), jnp.int32),
        pltpu.VMEM((2, BLK), jnp.int32),
        pltpu.VMEM((2, BLK), jnp.int32),
        pltpu.VMEM((2, BLK), jnp.int32),
        pltpu.VMEM((2 * BLK, C_DIM), jnp.float32),
        pltpu.VMEM((2 * BLK, C_DIM), jnp.float32),
        pltpu.VMEM((BLK, C_DIM), jnp.float32),
        pltpu.VMEM((64,), jnp.float32),
        pltpu.SemaphoreType.DMA,
        pltpu.SemaphoreType.DMA,
    ],
)
def _triplets(proj_hbm, x_hbm, aidx_hbm, r2_hbm, out_hbm,
              xv, idx0_v, idx1_v, g0_v, g1_v, buf0_v, buf1_v, out_v, r_v,
              sem0, sem1):
    wid = lax.axis_index("s") * NC + lax.axis_index("c")
    pltpu.sync_copy(r2_hbm, r_v)
    pltpu.sync_copy(x_hbm, xv.at[pl.ds(0, N_X)])
    p1 = (wid >= (NW // 2)).astype(jnp.int32)
    m = jnp.broadcast_to(p1.astype(jnp.float32), (16,))
    r_lo = r_v[pl.ds(0, 16)] + m * (r_v[pl.ds(32, 16)] - r_v[pl.ds(0, 16)])
    r_hi = r_v[pl.ds(16, 16)] + m * (r_v[pl.ds(48, 16)] - r_v[pl.ds(16, 16)])
    base = wid * BLOCKS_PER_W

    idx_s = (idx0_v, idx1_v)
    g_s = (g0_v, g1_v)
    buf_s = (buf0_v, buf1_v)
    sem_s = (sem0, sem1)

    def fire(block, slot):
        pltpu.sync_copy(aidx_hbm.at[block], idx_s[slot])
        gv = g_s[slot]
        iv = idx_s[slot]
        for r in range(2):
            for i in range(BLK // 16):
                a = iv[r, pl.ds(16 * i, 16)]
                g = plsc.load_gather(xv, [a])
                gp = (jnp.bitwise_and(g, -8192)
                      + jnp.left_shift(jnp.bitwise_and(g, 2047), 2)
                      + jnp.bitwise_and(jnp.right_shift(g, 11), 3))
                gv[r, pl.ds(16 * i, 16)] = gp
        pltpu.async_copy(proj_hbm.at[gv.at[0]],
                         buf_s[slot].at[pl.ds(0, BLK)], sem_s[slot])
        pltpu.async_copy(proj_hbm.at[gv.at[1]],
                         buf_s[slot].at[pl.ds(BLK, BLK)], sem_s[slot])

    def wait_slot(slot):
        pltpu.make_async_copy(proj_hbm.at[g_s[slot].at[0]],
                              buf_s[slot].at[pl.ds(0, BLK)], sem_s[slot]).wait()
        pltpu.make_async_copy(proj_hbm.at[g_s[slot].at[1]],
                              buf_s[slot].at[pl.ds(BLK, BLK)], sem_s[slot]).wait()

    def consume(b, slot):
        buf = buf_s[slot]
        for j in range(BLK):
            out_v[j, pl.ds(0, 16)] = (
                buf[j, pl.ds(0, 16)] - buf[BLK + j, pl.ds(0, 16)] + r_lo)
            out_v[j, pl.ds(16, 16)] = (
                buf[j, pl.ds(16, 16)] - buf[BLK + j, pl.ds(16, 16)] + r_hi)
        pltpu.sync_copy(out_v, out_hbm.at[pl.ds((base + b) * BLK, BLK)])

    fire(base, 0)

    def do_pair(s, _):
        b0 = 2 * s
        fire(base + b0 + 1, 1)
        wait_slot(0)
        consume(b0, 0)
        fire(base + jnp.minimum(b0 + 2, BLOCKS_PER_W - 1), 0)
        wait_slot(1)
        consume(b0 + 1, 1)
        return 0

    lax.fori_loop(0, BLOCKS_PER_W // 2, do_pair, 0)
    wait_slot(0)


def _as_blocks(a):
    return a.T.reshape(2, T_PER_PRED // BLK, BLK).transpose(1, 0, 2)


def kernel(X_domain, A_pred0, A_pred1, constant_table, predicate_table,
           W_c, b_c, W_p, b_p):
    del b_c
    ct_t = constant_table.T
    tail = jnp.pad(ct_t[:, _S1_NCOL * _S1_COLS:], ((0, 0), (0, _S1_COLS - _TAIL)))
    proj128, r2 = _project(ct_t, tail, W_c, predicate_table, W_p,
                           b_p.reshape(1, C_DIM))
    proj = proj128.reshape(PROJ_ROWS, C_DIM)
    aidx = jnp.concatenate([_as_blocks(A_pred0), _as_blocks(A_pred1)], axis=0)
    return _triplets(proj, X_domain.astype(jnp.int32), aidx, r2.reshape(64))


# speedup vs baseline: 1.5481x; 1.5481x over previous
import functools

import jax
import jax.numpy as jnp
from jax import lax
from jax.experimental import pallas as pl
from jax.experimental.pallas import tpu as pltpu
from jax.experimental.pallas import tpu_sc as plsc

NUM_CONSTANTS = 1000000
C_DIM = 32
N_X = 100000
T_PER_PRED = 65536
T = 2 * T_PER_PRED

NC, NS = 2, 16
NW = NC * NS

BLK = 128
N_BLOCKS = T // BLK
BLOCKS_PER_W = N_BLOCKS // NW
X_PAD = 100352

_MESH = plsc.VectorSubcoreMesh(
    core_axis_name="c", subcore_axis_name="s", num_cores=NC, num_subcores=NS)
_SC_PARAMS = pltpu.CompilerParams(
    use_tc_tiling_on_sc=False, needs_layout_passes=False)


_GROUP = 2048
_S1_COLS = 4 * _GROUP
_S1_NCOL = NUM_CONSTANTS // _S1_COLS
_S1_GRID = 123
_S1_OUTROWS = _S1_GRID * _GROUP
PROJ_ROWS = 4 * _S1_OUTROWS
_TAIL = NUM_CONSTANTS - _S1_NCOL * _S1_COLS


def _project_body(xt_ref, tail_ref, w_ref, p_ref, wp_ref, bp_ref, o_ref, r2_ref):
    g = pl.program_id(0)
    x = jnp.where(g >= _S1_NCOL, tail_ref[...], xt_ref[...])
    part = jax.lax.dot_general(
        x, w_ref[...], (((0,), (0,)), ((), ())),
        preferred_element_type=jnp.float32)
    for a in range(4):
        o_ref[:, 32 * a:32 * a + 32] = part[_GROUP * a:_GROUP * (a + 1), :]

    @pl.when(g == 0)
    def _():
        r2_ref[...] = (
            jnp.dot(p_ref[...], wp_ref[...], preferred_element_type=jnp.float32)
            + bp_ref[...]
        )


_project = pl.pallas_call(
    _project_body,
    grid=(_S1_GRID,),
    in_specs=[
        pl.BlockSpec((C_DIM, _S1_COLS),
                     lambda g: (0, jnp.minimum(g, _S1_NCOL - 1))),
        pl.BlockSpec((C_DIM, _S1_COLS), lambda g: (0, 0)),
        pl.BlockSpec((C_DIM, C_DIM), lambda g: (0, 0)),
        pl.BlockSpec((2, C_DIM), lambda g: (0, 0)),
        pl.BlockSpec((C_DIM, C_DIM), lambda g: (0, 0)),
        pl.BlockSpec((1, C_DIM), lambda g: (0, 0)),
    ],
    out_specs=[
        pl.BlockSpec((_GROUP, 128), lambda g: (g, 0)),
        pl.BlockSpec((2, C_DIM), lambda g: (0, 0)),
    ],
    out_shape=[
        jax.ShapeDtypeStruct((_S1_OUTROWS, 128), jnp.float32),
        jax.ShapeDtypeStruct((2, C_DIM), jnp.float32),
    ],
    compiler_params=pltpu.CompilerParams(fuse_transposed_lhs_in_matmul=True),
)


@functools.partial(
    pl.kernel,
    out_type=jax.ShapeDtypeStruct((T, C_DIM), jnp.float32),
    mesh=_MESH,
    compiler_params=_SC_PARAMS,
    scratch_types=[
        pltpu.VMEM((X_PAD,), jnp.int32),
        pltpu.VMEM((2, BLK), jnp.int32),
        pltpu.VMEM((2, BLK), jnp.int32),
        pltpu.VMEM((2, BLK), jnp.int32),
        pltpu.VMEM((2, BLK), jnp.int32),
        pltpu.VMEM((2 * BLK, C_DIM), jnp.float32),
        pltpu.VMEM((2 * BLK, C_DIM), jnp.float32),
        pltpu.VMEM((BLK, C_DIM), jnp.float32),
        pltpu.VMEM((64,), jnp.float32),
        pltpu.SemaphoreType.DMA,
        pltpu.SemaphoreType.DMA,
    ],
)
def _triplets(proj_hbm, x_hbm, aidx_hbm, r2_hbm, out_hbm,
              xv, idx0_v, idx1_v, g0_v, g1_v, buf0_v, buf1_v, out_v, r_v,
              sem0, sem1):
    wid = lax.axis_index("s") * NC + lax.axis_index("c")
    pltpu.sync_copy(r2_hbm, r_v)
    pltpu.sync_copy(x_hbm, xv.at[pl.ds(0, N_X)])
    p1 = (wid >= (NW // 2)).astype(jnp.int32)
    m = jnp.broadcast_to(p1.astype(jnp.float32), (16,))
    r_lo = r_v[pl.ds(0, 16)] + m * (r_v[pl.ds(32, 16)] - r_v[pl.ds(0, 16)])
    r_hi = r_v[pl.ds(16, 16)] + m * (r_v[pl.ds(48, 16)] - r_v[pl.ds(16, 16)])
    base = wid * BLOCKS_PER_W

    idx_s = (idx0_v, idx1_v)
    g_s = (g0_v, g1_v)
    buf_s = (buf0_v, buf1_v)
    sem_s = (sem0, sem1)

    def fire(block, slot):
        pltpu.sync_copy(aidx_hbm.at[block], idx_s[slot])
        gv = g_s[slot]
        iv = idx_s[slot]
        for r in range(2):
            for i in range(BLK // 16):
                a = iv[r, pl.ds(16 * i, 16)]
                g = plsc.load_gather(xv, [a])
                gp = (jnp.bitwise_and(g, -8192)
                      + jnp.left_shift(jnp.bitwise_and(g, 2047), 2)
                      + jnp.bitwise_and(jnp.right_shift(g, 11), 3))
                gv[r, pl.ds(16 * i, 16)] = gp
        pltpu.async_copy(proj_hbm.at[gv.at[0]],
                         buf_s[slot].at[pl.ds(0, BLK)], sem_s[slot])
        pltpu.async_copy(proj_hbm.at[gv.at[1]],
                         buf_s[slot].at[pl.ds(BLK, BLK)], sem_s[slot])

    def wait_slot(slot):
        pltpu.make_async_copy(proj_hbm.at[g_s[slot].at[0]],
                              buf_s[slot].at[pl.ds(0, BLK)], sem_s[slot]).wait()
        pltpu.make_async_copy(proj_hbm.at[g_s[slot].at[1]],
                              buf_s[slot].at[pl.ds(BLK, BLK)], sem_s[slot]).wait()

    def consume(b, slot):
        buf = buf_s[slot]
        for j in range(BLK):
            out_v[j, pl.ds(0, 16)] = (
                buf[j, pl.ds(0, 16)] - buf[BLK + j, pl.ds(0, 16)] + r_lo)
            out_v[j, pl.ds(16, 16)] = (
                buf[j, pl.ds(16, 16)] - buf[BLK + j, pl.ds(16, 16)] + r_hi)
        pltpu.sync_copy(out_v, out_hbm.at[pl.ds((base + b) * BLK, BLK)])

    fire(base, 0)

    def do_pair(s, _):
        b0 = 2 * s
        fire(base + b0 + 1, 1)
        wait_slot(0)
        consume(b0, 0)
        fire(base + jnp.minimum(b0 + 2, BLOCKS_PER_W - 1), 0)
        wait_slot(1)
        consume(b0 + 1, 1)
        return 0

    lax.fori_loop(0, BLOCKS_PER_W // 2, do_pair, 0)
    wait_slot(0)


def _as_blocks(a):
    return a.T.reshape(2, T_PER_PRED // BLK, BLK).transpose(1, 0, 2)


def kernel(X_domain, A_pred0, A_pred1, constant_table, predicate_table,
           W_c, b_c, W_p, b_p):
    del b_c
    ct_t = constant_table.T
    tail = jnp.pad(ct_t[:, _S1_NCOL * _S1_COLS:], ((0, 0), (0, _S1_COLS - _TAIL)))
    proj128, r2 = _project(ct_t, tail, W_c, predicate_table, W_p,
                           b_p.reshape(1, C_DIM))
    proj = proj128.reshape(PROJ_ROWS, C_DIM)
    aidx = jnp.concatenate([_as_blocks(A_pred0), _as_blocks(A_pred1)], axis=0)
    return _triplets(proj, X_domain.astype(jnp.int32), aidx, r2.reshape(64))


# speedup vs baseline: 1.5698x; 1.0141x over previous
import functools

import jax
import jax.numpy as jnp
from jax import lax
from jax.experimental import pallas as pl
from jax.experimental.pallas import tpu as pltpu
from jax.experimental.pallas import tpu_sc as plsc

NUM_CONSTANTS = 1000000
C_DIM = 32
N_X = 100000
T_PER_PRED = 65536
T = 2 * T_PER_PRED

NC, NS = 2, 16
NW = NC * NS

BLK = 128
N_BLOCKS = T // BLK
BLOCKS_PER_W = N_BLOCKS // NW
X_PAD = 100352

_MESH = plsc.VectorSubcoreMesh(
    core_axis_name="c", subcore_axis_name="s", num_cores=NC, num_subcores=NS)
_SC_PARAMS = pltpu.CompilerParams(
    use_tc_tiling_on_sc=False, needs_layout_passes=False)


_GROUP = 4096
_S1_COLS = 4 * _GROUP
_S1_NCOL = NUM_CONSTANTS // _S1_COLS
_S1_GRID = 62
_S1_OUTROWS = _S1_GRID * _GROUP
PROJ_ROWS = 4 * _S1_OUTROWS
_TAIL = NUM_CONSTANTS - _S1_NCOL * _S1_COLS


def _project_body(xt_ref, tail_ref, w_ref, p_ref, wp_ref, bp_ref, o_ref, r2_ref):
    g = pl.program_id(0)
    x = jnp.where(g >= _S1_NCOL, tail_ref[...], xt_ref[...])
    part = jax.lax.dot_general(
        x, w_ref[...], (((0,), (0,)), ((), ())),
        preferred_element_type=jnp.float32)
    for a in range(4):
        o_ref[:, 32 * a:32 * a + 32] = part[_GROUP * a:_GROUP * (a + 1), :]

    @pl.when(g == 0)
    def _():
        r2_ref[...] = (
            jnp.dot(p_ref[...], wp_ref[...], preferred_element_type=jnp.float32)
            + bp_ref[...]
        )


_project = pl.pallas_call(
    _project_body,
    grid=(_S1_GRID,),
    in_specs=[
        pl.BlockSpec((C_DIM, _S1_COLS),
                     lambda g: (0, jnp.minimum(g, _S1_NCOL - 1))),
        pl.BlockSpec((C_DIM, _S1_COLS), lambda g: (0, 0)),
        pl.BlockSpec((C_DIM, C_DIM), lambda g: (0, 0)),
        pl.BlockSpec((2, C_DIM), lambda g: (0, 0)),
        pl.BlockSpec((C_DIM, C_DIM), lambda g: (0, 0)),
        pl.BlockSpec((1, C_DIM), lambda g: (0, 0)),
    ],
    out_specs=[
        pl.BlockSpec((_GROUP, 128), lambda g: (g, 0)),
        pl.BlockSpec((2, C_DIM), lambda g: (0, 0)),
    ],
    out_shape=[
        jax.ShapeDtypeStruct((_S1_OUTROWS, 128), jnp.float32),
        jax.ShapeDtypeStruct((2, C_DIM), jnp.float32),
    ],
    compiler_params=pltpu.CompilerParams(fuse_transposed_lhs_in_matmul=True),
)


@functools.partial(
    pl.kernel,
    out_type=jax.ShapeDtypeStruct((T, C_DIM), jnp.float32),
    mesh=_MESH,
    compiler_params=_SC_PARAMS,
    scratch_types=[
        pltpu.VMEM((X_PAD,), jnp.int32),
        pltpu.VMEM((2, BLK), jnp.int32),
        pltpu.VMEM((2, BLK), jnp.int32),
        pltpu.VMEM((2, BLK), jnp.int32),
        pltpu.VMEM((2, BLK), jnp.int32),
        pltpu.VMEM((2 * BLK, C_DIM), jnp.float32),
        pltpu.VMEM((2 * BLK, C_DIM), jnp.float32),
        pltpu.VMEM((BLK, C_DIM), jnp.float32),
        pltpu.VMEM((64,), jnp.float32),
        pltpu.SemaphoreType.DMA,
        pltpu.SemaphoreType.DMA,
    ],
)
def _triplets(proj_hbm, x_hbm, aidx_hbm, r2_hbm, out_hbm,
              xv, idx0_v, idx1_v, g0_v, g1_v, buf0_v, buf1_v, out_v, r_v,
              sem0, sem1):
    wid = lax.axis_index("s") * NC + lax.axis_index("c")
    pltpu.sync_copy(r2_hbm, r_v)
    pltpu.sync_copy(x_hbm, xv.at[pl.ds(0, N_X)])
    p1 = (wid >= (NW // 2)).astype(jnp.int32)
    m = jnp.broadcast_to(p1.astype(jnp.float32), (16,))
    r_lo = r_v[pl.ds(0, 16)] + m * (r_v[pl.ds(32, 16)] - r_v[pl.ds(0, 16)])
    r_hi = r_v[pl.ds(16, 16)] + m * (r_v[pl.ds(48, 16)] - r_v[pl.ds(16, 16)])
    base = wid * BLOCKS_PER_W

    idx_s = (idx0_v, idx1_v)
    g_s = (g0_v, g1_v)
    buf_s = (buf0_v, buf1_v)
    sem_s = (sem0, sem1)

    def fire(block, slot):
        pltpu.sync_copy(aidx_hbm.at[block], idx_s[slot])
        gv = g_s[slot]
        iv = idx_s[slot]
        for r in range(2):
            for i in range(BLK // 16):
                a = iv[r, pl.ds(16 * i, 16)]
                g = plsc.load_gather(xv, [a])
                gp = (jnp.bitwise_and(g, -_S1_COLS)
                      + jnp.left_shift(jnp.bitwise_and(g, _GROUP - 1), 2)
                      + jnp.bitwise_and(jnp.right_shift(g, 12), 3))
                gv[r, pl.ds(16 * i, 16)] = gp
        pltpu.async_copy(proj_hbm.at[gv.at[0]],
                         buf_s[slot].at[pl.ds(0, BLK)], sem_s[slot])
        pltpu.async_copy(proj_hbm.at[gv.at[1]],
                         buf_s[slot].at[pl.ds(BLK, BLK)], sem_s[slot])

    def wait_slot(slot):
        pltpu.make_async_copy(proj_hbm.at[g_s[slot].at[0]],
                              buf_s[slot].at[pl.ds(0, BLK)], sem_s[slot]).wait()
        pltpu.make_async_copy(proj_hbm.at[g_s[slot].at[1]],
                              buf_s[slot].at[pl.ds(BLK, BLK)], sem_s[slot]).wait()

    def consume(b, slot):
        buf = buf_s[slot]
        for j in range(BLK):
            out_v[j, pl.ds(0, 16)] = (
                buf[j, pl.ds(0, 16)] - buf[BLK + j, pl.ds(0, 16)] + r_lo)
            out_v[j, pl.ds(16, 16)] = (
                buf[j, pl.ds(16, 16)] - buf[BLK + j, pl.ds(16, 16)] + r_hi)
        pltpu.sync_copy(out_v, out_hbm.at[pl.ds((base + b) * BLK, BLK)])

    fire(base, 0)

    def do_pair(s, _):
        b0 = 2 * s
        fire(base + b0 + 1, 1)
        wait_slot(0)
        consume(b0, 0)
        fire(base + jnp.minimum(b0 + 2, BLOCKS_PER_W - 1), 0)
        wait_slot(1)
        consume(b0 + 1, 1)
        return 0

    lax.fori_loop(0, BLOCKS_PER_W // 2, do_pair, 0)
    wait_slot(0)


def _as_blocks(a):
    return a.T.reshape(2, T_PER_PRED // BLK, BLK).transpose(1, 0, 2)


def kernel(X_domain, A_pred0, A_pred1, constant_table, predicate_table,
           W_c, b_c, W_p, b_p):
    del b_c
    ct_t = constant_table.T
    tail = jnp.pad(ct_t[:, _S1_NCOL * _S1_COLS:], ((0, 0), (0, _S1_COLS - _TAIL)))
    proj128, r2 = _project(ct_t, tail, W_c, predicate_table, W_p,
                           b_p.reshape(1, C_DIM))
    proj = proj128.reshape(PROJ_ROWS, C_DIM)
    aidx = jnp.concatenate([_as_blocks(A_pred0), _as_blocks(A_pred1)], axis=0)
    return _triplets(proj, X_domain.astype(jnp.int32), aidx, r2.reshape(64))
